# baseline (speedup 1.0000x reference)
import functools

import jax
import jax.numpy as jnp
from jax import lax
from jax.experimental import pallas as pl
from jax.experimental.pallas import tpu as pltpu
from jax.experimental.pallas import tpu_sc as plsc

B, NF, NT, V, D, L = 4096, 10, 26, 100000, 32, 20
DENSE_OUT, OVER_OUT = 32, 16

_NC = 2
_NS = 16
_NW = _NC * _NS

_BPW = B // _NW
_CP = 32
_NCB = _BPW // _CP
_NU = NT * _NCB
_SL = 128
_RS = _CP * L // _SL


@functools.cache
def _get_sc_pool():
    mesh = plsc.VectorSubcoreMesh(core_axis_name="c", subcore_axis_name="s")
    return functools.partial(
        pl.kernel,
        mesh=mesh,
        compiler_params=pltpu.CompilerParams(use_tc_tiling_on_sc=False),
        out_type=jax.ShapeDtypeStruct((NT, B, D), jnp.float32),
        scratch_types=[
            pltpu.VMEM((2, _RS, _SL), jnp.int32),
            pltpu.VMEM((2, _RS, _SL, D), jnp.float32),
            pltpu.VMEM((2, _CP, D), jnp.float32),
            pltpu.SemaphoreType.DMA,
            pltpu.SemaphoreType.DMA,
        ],
    )(_sc_pool_body)


def _sc_pool_body(tables_hbm, idx_hbm, out_hbm, idx_v, rows_v, out_v, sem0, sem1):
    wid = lax.axis_index("s") * _NC + lax.axis_index("c")
    b0 = wid * _BPW
    sems = (sem0, sem1)

    def start(slot, u, sem):
        t = u >> 2
        c = u & 3
        pltpu.sync_copy(idx_hbm.at[t, wid, c], idx_v.at[slot])
        for j in range(_RS):
            pltpu.async_copy(tables_hbm.at[t].at[idx_v.at[slot, j]],
                             rows_v.at[slot, j], sem)

    def drain(slot, u, sem):
        t = u >> 2
        for j in range(_RS):
            pltpu.make_async_copy(tables_hbm.at[t].at[idx_v.at[slot, j]],
                                  rows_v.at[slot, j], sem).wait()

    def compute(slot):
        def pair_body(p, carry):
            base = p * L
            acc_a = rows_v[slot, base >> 7, base & 127, pl.ds(0, 16)]
            acc_b = rows_v[slot, base >> 7, base & 127, pl.ds(16, 16)]
            for l in range(1, L):
                r = base + l
                j = r >> 7
                k = r & 127
                acc_a = acc_a + rows_v[slot, j, k, pl.ds(0, 16)]
                acc_b = acc_b + rows_v[slot, j, k, pl.ds(16, 16)]
            out_v[slot, p, pl.ds(0, 16)] = acc_a
            out_v[slot, p, pl.ds(16, 16)] = acc_b
            return carry

        lax.fori_loop(0, _CP, pair_body, 0)

    start(0, 0, sems[0])
    start(1, 1, sems[1])

    def loop_body(u2, carry):
        for slot in range(2):
            u = u2 * 2 + slot
            drain(slot, u, sems[slot])
            compute(slot)
            pltpu.sync_copy(
                out_v.at[slot],
                out_hbm.at[u >> 2, pl.ds(b0 + (u & 3) * _CP, _CP)])
            nxt = u + 2

            @pl.when(nxt < _NU)
            def _():
                start(slot, nxt, sems[slot])
        return carry

    lax.fori_loop(0, _NU // 2, loop_body, 0)


_BM = 512


def _head_body(ff, dw, db, pooled, owd, ows, ob, o):
    dense = jnp.maximum(
        jnp.dot(ff[:], dw[:], preferred_element_type=jnp.float32) + db[:], 0.0)
    acc = jnp.dot(dense, owd[:], preferred_element_type=jnp.float32) + ob[:]
    for t in range(NT):
        acc = acc + jnp.dot(pooled[t], ows[t],
                            preferred_element_type=jnp.float32)
    o[:] = acc


_tc_head = pl.pallas_call(
    _head_body,
    grid=(B // _BM,),
    in_specs=[
        pl.BlockSpec((_BM, NF), lambda i: (i, 0)),
        pl.BlockSpec((NF, DENSE_OUT), lambda i: (0, 0)),
        pl.BlockSpec((1, DENSE_OUT), lambda i: (0, 0)),
        pl.BlockSpec((NT, _BM, D), lambda i: (0, i, 0)),
        pl.BlockSpec((DENSE_OUT, OVER_OUT), lambda i: (0, 0)),
        pl.BlockSpec((NT, D, OVER_OUT), lambda i: (0, 0, 0)),
        pl.BlockSpec((1, OVER_OUT), lambda i: (0, 0)),
    ],
    out_specs=pl.BlockSpec((_BM, OVER_OUT), lambda i: (i, 0)),
    out_shape=jax.ShapeDtypeStruct((B, OVER_OUT), jnp.float32),
)


def kernel(float_features, indices, tables, dense_w, dense_b, over_w, over_b):
    idx_t = jnp.transpose(indices.astype(jnp.int32), (1, 0, 2))
    idx_chunks = idx_t.reshape(NT, _NW, _NCB, _RS, _SL)
    pooled = _get_sc_pool()(tables, idx_chunks)
    ows = over_w[DENSE_OUT:].reshape(NT, D, OVER_OUT)
    out = _tc_head(float_features, dense_w, dense_b.reshape(1, DENSE_OUT),
                   pooled, over_w[:DENSE_OUT], ows,
                   over_b.reshape(1, OVER_OUT))
    return out
